# Initial kernel scaffold; baseline (speedup 1.0000x reference)
import functools

import numpy as np
import jax
import jax.numpy as jnp
from jax import lax
from jax.experimental import pallas as pl
from jax.experimental.pallas import tpu as pltpu
from jax.experimental.pallas import tpu_sc as plsc

N = 10000
C = 128
E = 320000
GROUPS = 8
GSIZE = C // GROUPS
EPS = 1e-5

NCORES = 2
NSUB = 16
NW = NCORES * NSUB
CHUNK = 128
EPW = 10240
NCH = EPW // CHUNK
E_PAD = EPW * NW
ROWS_PER_SUB = N // NSUB
ZCHUNK = 125


def _edge_body(h_hbm, src_hbm, dst_hbm, ew_hbm, out_hbm,
               src_v, dst_v, ew_v, rows, zbuf, table, sem):
    c = lax.axis_index("c")
    s = lax.axis_index("s")
    wid = c * NSUB + s

    pltpu.sync_copy(src_hbm.at[wid], src_v)
    pltpu.sync_copy(dst_hbm.at[wid], dst_v)
    pltpu.sync_copy(ew_hbm.at[wid], ew_v)

    zv = jnp.zeros((16,), jnp.float32)

    def zrow(i, carry):
        for jj in range(GROUPS):
            zbuf[i, pl.ds(jj * 16, 16)] = zv
        return carry

    lax.fori_loop(0, ZCHUNK, zrow, 0)
    for k in range(ROWS_PER_SUB // ZCHUNK):
        pltpu.sync_copy(zbuf, table.at[pl.ds(s * ROWS_PER_SUB + k * ZCHUNK, ZCHUNK)])
    plsc.subcore_barrier()

    def chunk(j, carry):
        pltpu.async_copy(h_hbm.at[src_v.at[j]], rows, sem).wait()

        def edge(i, ecarry):
            w = ew_v[j, i]
            for sub in range(GROUPS):
                sl = pl.ds(sub * 16, 16)
                rows[i, sl] = rows[i, sl] * w
            return ecarry

        lax.fori_loop(0, CHUNK, edge, 0)
        pltpu.sync_copy(rows, table.at[dst_v.at[j]], add=True)
        return carry

    lax.fori_loop(0, NCH, chunk, 0)
    plsc.subcore_barrier()

    pltpu.sync_copy(table.at[pl.ds(s * ROWS_PER_SUB, ROWS_PER_SUB)],
                    out_hbm.at[pl.ds(c * N + s * ROWS_PER_SUB, ROWS_PER_SUB)])


_edge_pass = functools.partial(
    pl.kernel,
    out_type=jax.ShapeDtypeStruct((NCORES * N, C), jnp.float32),
    mesh=plsc.VectorSubcoreMesh(core_axis_name="c", subcore_axis_name="s"),
    scratch_types=[
        pltpu.VMEM((NCH, CHUNK), jnp.int32),
        pltpu.VMEM((NCH, CHUNK), jnp.int32),
        pltpu.VMEM((NCH, CHUNK), jnp.float32),
        pltpu.VMEM((CHUNK, C), jnp.float32),
        pltpu.VMEM((ZCHUNK, C), jnp.float32),
        pltpu.VMEM_SHARED((N, C), jnp.float32),
        pltpu.SemaphoreType.DMA,
    ],
)(_edge_body)


RBLK = 1000
GRID = N // RBLK

_MAVG = np.kron(np.eye(GROUPS, dtype=np.float32),
                np.ones((GSIZE, GSIZE), dtype=np.float32) / GSIZE)


def _leaky(x):
    return jnp.where(x >= 0, x, 0.01 * x)


def _make_dense_call(n_in, use_gn, use_mm, want_sum, w_cols=C):
    def body(*refs):
        ins = refs[:n_in]
        k = n_in
        bias = refs[k][...]
        k += 1
        if use_gn:
            gamma = refs[k][...]; beta = refs[k + 1][...]; mavg = refs[k + 2][...]
            k += 3
        if use_mm:
            w = refs[k][...]
            k += 1
        outs = refs[k:]
        s = ins[0][...]
        for r in ins[1:]:
            s = s + r[...]
        s = s + bias
        if want_sum:
            outs[-1][...] = s
        if use_gn:
            m = jnp.dot(s, mavg, preferred_element_type=jnp.float32)
            xc = s - m
            var = jnp.dot(xc * xc, mavg, preferred_element_type=jnp.float32)
            y = xc * lax.rsqrt(var + EPS) * gamma + beta
            y = _leaky(y)
        else:
            y = s
        if use_mm:
            outs[0][...] = jnp.dot(y, w, preferred_element_type=jnp.float32)
        elif not want_sum:
            outs[0][...] = y

    def call(inputs, bias, gn=None, w=None):
        in_specs = [pl.BlockSpec((RBLK, C), lambda i, o=off: (i + o, 0))
                    for (_, off) in inputs]
        args = [a for (a, _) in inputs]
        args.append(bias.reshape(1, -1))
        in_specs.append(pl.BlockSpec((1, C), lambda i: (0, 0)))
        if use_gn:
            gamma, beta = gn
            args += [gamma.reshape(1, -1), beta.reshape(1, -1),
                     jnp.asarray(_MAVG)]
            in_specs += [pl.BlockSpec((1, C), lambda i: (0, 0)),
                         pl.BlockSpec((1, C), lambda i: (0, 0)),
                         pl.BlockSpec((C, C), lambda i: (0, 0))]
        if use_mm:
            args.append(w)
            in_specs.append(pl.BlockSpec((C, w_cols), lambda i: (0, 0)))
        out_shapes = []
        out_specs = []
        if use_mm or not want_sum:
            oc = w_cols if use_mm else C
            out_shapes.append(jax.ShapeDtypeStruct((N, oc), jnp.float32))
            out_specs.append(pl.BlockSpec((RBLK, oc), lambda i: (i, 0)))
        if want_sum:
            out_shapes.append(jax.ShapeDtypeStruct((N, C), jnp.float32))
            out_specs.append(pl.BlockSpec((RBLK, C), lambda i: (i, 0)))
        return pl.pallas_call(
            body,
            grid=(GRID,),
            in_specs=in_specs,
            out_specs=out_specs if len(out_specs) > 1 else out_specs[0],
            out_shape=tuple(out_shapes) if len(out_shapes) > 1 else out_shapes[0],
        )(*args)

    return call


def _t_embed_body(t_ref, w_ref, b_ref, o_ref):
    lt = _leaky(t_ref[...])
    o_ref[...] = jnp.dot(lt, w_ref[...],
                         preferred_element_type=jnp.float32) + b_ref[...]


def _t_embed(t, wcat, bcat):
    return pl.pallas_call(
        _t_embed_body,
        grid=(GRID,),
        in_specs=[pl.BlockSpec((RBLK, C), lambda i: (i, 0)),
                  pl.BlockSpec((C, 4 * C), lambda i: (0, 0)),
                  pl.BlockSpec((1, 4 * C), lambda i: (0, 0))],
        out_specs=pl.BlockSpec((RBLK, 4 * C), lambda i: (i, 0)),
        out_shape=jax.ShapeDtypeStruct((N, 4 * C), jnp.float32),
    )(t, wcat, bcat.reshape(1, -1))


def kernel(x, t, edge_index, edge_weight, params):
    src = edge_index[0].astype(jnp.int32)
    dst = edge_index[1].astype(jnp.int32)
    pad = E_PAD - E
    src_p = jnp.concatenate([src, jnp.zeros((pad,), jnp.int32)]).reshape(NW, NCH, CHUNK)
    dst_p = jnp.concatenate([dst, jnp.zeros((pad,), jnp.int32)]).reshape(NW, NCH, CHUNK)
    ew_p = jnp.concatenate([edge_weight.astype(jnp.float32),
                            jnp.zeros((pad,), jnp.float32)]).reshape(NW, NCH, CHUNK)

    wtcat = jnp.concatenate([p['Wt'] for p in params], axis=1)
    btcat = jnp.concatenate([p['bt'] for p in params])
    tts = _t_embed(t, wtcat, btcat)
    tt = [lax.slice(tts, (0, b * C), (N, (b + 1) * C)) for b in range(4)]

    gn_mm_1 = _make_dense_call(1, True, True, False)
    gn_mm_3 = _make_dense_call(3, True, True, False)
    gn_mm_3s = _make_dense_call(3, True, True, True)
    gn_mm_4s = _make_dense_call(4, True, True, True)
    sum_3 = _make_dense_call(3, False, False, False)

    def econv(h):
        return _edge_pass(h, src_p, dst_p, ew_p)

    zb = jnp.zeros((C,), jnp.float32)
    p0, p1, p2, p3 = params

    u1 = gn_mm_1([(x, 0)], zb, gn=(p0['gn1_g'], p0['gn1_b']), w=p0['W1'])
    P1 = econv(u1)
    v1 = gn_mm_3([(P1, 0), (P1, GRID), (tt[0], 0)], p0['b1'],
                 gn=(p0['gn2_g'], p0['gn2_b']), w=p0['W2'])
    Q1 = econv(v1)

    u2, h1 = gn_mm_3s([(Q1, 0), (Q1, GRID), (x, 0)], p0['b2'],
                      gn=(p1['gn1_g'], p1['gn1_b']), w=p1['W1'])
    P2 = econv(u2)
    v2 = gn_mm_3([(P2, 0), (P2, GRID), (tt[1], 0)], p1['b1'],
                 gn=(p1['gn2_g'], p1['gn2_b']), w=p1['W2'])
    Q2 = econv(v2)

    u3, h2 = gn_mm_3s([(Q2, 0), (Q2, GRID), (h1, 0)], p1['b2'],
                      gn=(p2['gn1_g'], p2['gn1_b']), w=p2['W1'])
    P3 = econv(u3)
    v3 = gn_mm_3([(P3, 0), (P3, GRID), (tt[2], 0)], p2['b1'],
                 gn=(p2['gn2_g'], p2['gn2_b']), w=p2['W2'])
    Q3 = econv(v3)

    u4, s4 = gn_mm_4s([(Q3, 0), (Q3, GRID), (h2, 0), (h1, 0)], p2['b2'],
                      gn=(p3['gn1_g'], p3['gn1_b']), w=p3['W1'])
    P4 = econv(u4)
    v4 = gn_mm_3([(P4, 0), (P4, GRID), (tt[3], 0)], p3['b1'],
                 gn=(p3['gn2_g'], p3['gn2_b']), w=p3['W2'])
    Q4 = econv(v4)

    return sum_3([(Q4, 0), (Q4, GRID), (s4, 0)], p3['b2'])


# speedup vs baseline: 2.6469x; 2.6469x over previous
import functools

import numpy as np
import jax
import jax.numpy as jnp
from jax import lax
from jax.experimental import pallas as pl
from jax.experimental.pallas import tpu as pltpu
from jax.experimental.pallas import tpu_sc as plsc

N = 10000
C = 128
E = 320000
GROUPS = 8
GSIZE = C // GROUPS
EPS = 1e-5

NCORES = 2
NSUB = 16
NW = NCORES * NSUB
CHUNK = 128
EPW = 10240
NCH = EPW // CHUNK
E_PAD = EPW * NW
ROWS_PER_SUB = N // NSUB
ZCHUNK = 125


def _edge_body(h_hbm, src_hbm, dst_hbm, ew_hbm, out_hbm,
               src_v, dst_v, ew_v, rows, table, sem):
    c = lax.axis_index("c")
    s = lax.axis_index("s")
    wid = c * NSUB + s

    pltpu.sync_copy(src_hbm.at[wid], src_v)
    pltpu.sync_copy(dst_hbm.at[wid], dst_v)
    pltpu.sync_copy(ew_hbm.at[wid], ew_v)

    zv = jnp.zeros((16,), jnp.float32)

    def zrow(i, carry):
        for jj in range(GROUPS):
            rows[i, pl.ds(jj * 16, 16)] = zv
        return carry

    lax.fori_loop(0, CHUNK, zrow, 0)
    for k in range(4):
        pltpu.sync_copy(rows, table.at[pl.ds(s * ROWS_PER_SUB + k * CHUNK, CHUNK)])
    pltpu.sync_copy(rows.at[pl.ds(0, ROWS_PER_SUB - 4 * CHUNK)],
                    table.at[pl.ds(s * ROWS_PER_SUB + 4 * CHUNK,
                                   ROWS_PER_SUB - 4 * CHUNK)])
    plsc.subcore_barrier()

    def chunk(j, carry):
        pltpu.async_copy(h_hbm.at[src_v.at[j]], rows, sem).wait()

        def egroup(g, ecarry):
            wv = ew_v[j, pl.ds(g * 16, 16)]
            for lane in range(16):
                w = wv[lane]
                i = g * 16 + lane
                for sub in range(GROUPS):
                    sl = pl.ds(sub * 16, 16)
                    rows[i, sl] = rows[i, sl] * w
            return ecarry

        lax.fori_loop(0, CHUNK // 16, egroup, 0)
        pltpu.sync_copy(rows, table.at[dst_v.at[j]], add=True)
        return carry

    lax.fori_loop(0, NCH, chunk, 0)
    plsc.subcore_barrier()

    @pl.when(s < NSUB - 1)
    def _():
        pltpu.sync_copy(table.at[pl.ds(s * 624, 624)],
                        out_hbm.at[pl.ds(c * N + s * 624, 624)])

    @pl.when(s == NSUB - 1)
    def _():
        pltpu.sync_copy(table.at[pl.ds(15 * 624, 640)],
                        out_hbm.at[pl.ds(c * N + 15 * 624, 640)])


@functools.lru_cache(maxsize=1)
def _build_edge_pass():
    return functools.partial(
        pl.kernel,
        out_type=jax.ShapeDtypeStruct((NCORES * N, C), jnp.float32),
        mesh=plsc.VectorSubcoreMesh(core_axis_name="c", subcore_axis_name="s"),
        scratch_types=[
            pltpu.VMEM((NCH, CHUNK), jnp.int32),
            pltpu.VMEM((NCH, CHUNK), jnp.int32),
            pltpu.VMEM((NCH, CHUNK), jnp.float32),
            pltpu.VMEM((CHUNK, C), jnp.float32),
            pltpu.VMEM_SHARED((N, C), jnp.float32),
            pltpu.SemaphoreType.DMA,
        ],
    )(_edge_body)


def _edge_pass(h, src_p, dst_p, ew_p):
    return _build_edge_pass()(h, src_p, dst_p, ew_p)


RBLK = 1000
GRID = N // RBLK

_MAVG = np.kron(np.eye(GROUPS, dtype=np.float32),
                np.ones((GSIZE, GSIZE), dtype=np.float32) / GSIZE)


def _leaky(x):
    return jnp.where(x >= 0, x, 0.01 * x)


def _make_dense_call(n_in, use_gn, use_mm, want_sum, w_cols=C):
    def body(*refs):
        ins = refs[:n_in]
        k = n_in
        bias = refs[k][...]
        k += 1
        if use_gn:
            gamma = refs[k][...]; beta = refs[k + 1][...]; mavg = refs[k + 2][...]
            k += 3
        if use_mm:
            w = refs[k][...]
            k += 1
        outs = refs[k:]
        s = ins[0][...]
        for r in ins[1:]:
            s = s + r[...]
        s = s + bias
        if want_sum:
            outs[-1][...] = s
        if use_gn:
            m = jnp.dot(s, mavg, preferred_element_type=jnp.float32)
            xc = s - m
            var = jnp.dot(xc * xc, mavg, preferred_element_type=jnp.float32)
            y = xc * lax.rsqrt(var + EPS) * gamma + beta
            y = _leaky(y)
        else:
            y = s
        if use_mm:
            outs[0][...] = jnp.dot(y, w, preferred_element_type=jnp.float32)
        elif not want_sum:
            outs[0][...] = y

    def call(inputs, bias, gn=None, w=None):
        in_specs = [pl.BlockSpec((RBLK, C), lambda i, o=off: (i + o, 0))
                    for (_, off) in inputs]
        args = [a for (a, _) in inputs]
        args.append(bias.reshape(1, -1))
        in_specs.append(pl.BlockSpec((1, C), lambda i: (0, 0)))
        if use_gn:
            gamma, beta = gn
            args += [gamma.reshape(1, -1), beta.reshape(1, -1),
                     jnp.asarray(_MAVG)]
            in_specs += [pl.BlockSpec((1, C), lambda i: (0, 0)),
                         pl.BlockSpec((1, C), lambda i: (0, 0)),
                         pl.BlockSpec((C, C), lambda i: (0, 0))]
        if use_mm:
            args.append(w)
            in_specs.append(pl.BlockSpec((C, w_cols), lambda i: (0, 0)))
        out_shapes = []
        out_specs = []
        if use_mm or not want_sum:
            oc = w_cols if use_mm else C
            out_shapes.append(jax.ShapeDtypeStruct((N, oc), jnp.float32))
            out_specs.append(pl.BlockSpec((RBLK, oc), lambda i: (i, 0)))
        if want_sum:
            out_shapes.append(jax.ShapeDtypeStruct((N, C), jnp.float32))
            out_specs.append(pl.BlockSpec((RBLK, C), lambda i: (i, 0)))
        return pl.pallas_call(
            body,
            grid=(GRID,),
            in_specs=in_specs,
            out_specs=out_specs if len(out_specs) > 1 else out_specs[0],
            out_shape=tuple(out_shapes) if len(out_shapes) > 1 else out_shapes[0],
        )(*args)

    return call


def _t_embed_body(t_ref, w_ref, b_ref, o_ref):
    lt = _leaky(t_ref[...])
    o_ref[...] = jnp.dot(lt, w_ref[...],
                         preferred_element_type=jnp.float32) + b_ref[...]


def _t_embed(t, wcat, bcat):
    return pl.pallas_call(
        _t_embed_body,
        grid=(GRID,),
        in_specs=[pl.BlockSpec((RBLK, C), lambda i: (i, 0)),
                  pl.BlockSpec((C, 4 * C), lambda i: (0, 0)),
                  pl.BlockSpec((1, 4 * C), lambda i: (0, 0))],
        out_specs=pl.BlockSpec((RBLK, 4 * C), lambda i: (i, 0)),
        out_shape=jax.ShapeDtypeStruct((N, 4 * C), jnp.float32),
    )(t, wcat, bcat.reshape(1, -1))


def kernel(x, t, edge_index, edge_weight, params):
    src = edge_index[0].astype(jnp.int32)
    dst = edge_index[1].astype(jnp.int32)
    pad = E_PAD - E
    src_p = jnp.concatenate([src, jnp.zeros((pad,), jnp.int32)]).reshape(NW, NCH, CHUNK)
    dst_p = jnp.concatenate([dst, jnp.zeros((pad,), jnp.int32)]).reshape(NW, NCH, CHUNK)
    ew_p = jnp.concatenate([edge_weight.astype(jnp.float32),
                            jnp.zeros((pad,), jnp.float32)]).reshape(NW, NCH, CHUNK)

    wtcat = jnp.concatenate([p['Wt'] for p in params], axis=1)
    btcat = jnp.concatenate([p['bt'] for p in params])
    tts = _t_embed(t, wtcat, btcat)
    tt = [lax.slice(tts, (0, b * C), (N, (b + 1) * C)) for b in range(4)]

    gn_mm_1 = _make_dense_call(1, True, True, False)
    gn_mm_3 = _make_dense_call(3, True, True, False)
    gn_mm_3s = _make_dense_call(3, True, True, True)
    gn_mm_4s = _make_dense_call(4, True, True, True)
    sum_3 = _make_dense_call(3, False, False, False)

    def econv(h):
        return _edge_pass(h, src_p, dst_p, ew_p)

    zb = jnp.zeros((C,), jnp.float32)
    p0, p1, p2, p3 = params

    u1 = gn_mm_1([(x, 0)], zb, gn=(p0['gn1_g'], p0['gn1_b']), w=p0['W1'])
    P1 = econv(u1)
    v1 = gn_mm_3([(P1, 0), (P1, GRID), (tt[0], 0)], p0['b1'],
                 gn=(p0['gn2_g'], p0['gn2_b']), w=p0['W2'])
    Q1 = econv(v1)

    u2, h1 = gn_mm_3s([(Q1, 0), (Q1, GRID), (x, 0)], p0['b2'],
                      gn=(p1['gn1_g'], p1['gn1_b']), w=p1['W1'])
    P2 = econv(u2)
    v2 = gn_mm_3([(P2, 0), (P2, GRID), (tt[1], 0)], p1['b1'],
                 gn=(p1['gn2_g'], p1['gn2_b']), w=p1['W2'])
    Q2 = econv(v2)

    u3, h2 = gn_mm_3s([(Q2, 0), (Q2, GRID), (h1, 0)], p1['b2'],
                      gn=(p2['gn1_g'], p2['gn1_b']), w=p2['W1'])
    P3 = econv(u3)
    v3 = gn_mm_3([(P3, 0), (P3, GRID), (tt[2], 0)], p2['b1'],
                 gn=(p2['gn2_g'], p2['gn2_b']), w=p2['W2'])
    Q3 = econv(v3)

    u4, s4 = gn_mm_4s([(Q3, 0), (Q3, GRID), (h2, 0), (h1, 0)], p2['b2'],
                      gn=(p3['gn1_g'], p3['gn1_b']), w=p3['W1'])
    P4 = econv(u4)
    v4 = gn_mm_3([(P4, 0), (P4, GRID), (tt[3], 0)], p3['b1'],
                 gn=(p3['gn2_g'], p3['gn2_b']), w=p3['W2'])
    Q4 = econv(v4)

    return sum_3([(Q4, 0), (Q4, GRID), (s4, 0)], p3['b2'])
